# baseline (speedup 1.0000x reference)
import jax
import jax.numpy as jnp
from jax.experimental import pallas as pl
from jax.experimental.pallas import tpu as pltpu

PLACEHOLDER = -1
TINY = float(jnp.finfo(jnp.float32).tiny)


def _body(temp_s, tok_s, bonus_s, tl_ref, dp_ref, q_ref, tok_ref, u_ref,
          out_ref):
    L, V = tl_ref.shape
    b = pl.program_id(0)
    temp = temp_s[b]
    tl = tl_ref[...]
    scaled = tl / temp
    m = jnp.max(scaled, axis=1, keepdims=True)
    e = jnp.exp(scaled - m)
    s = jnp.sum(e, axis=1, keepdims=True)

    dp = dp_ref[0]
    iota_v = jax.lax.broadcasted_iota(jnp.int32, (L, V), 1)
    tok_col = tok_ref[b]
    mask = iota_v == tok_col
    e_tok = jnp.sum(jnp.where(mask, e, 0.0), axis=1, keepdims=True)
    dp_tok = jnp.sum(jnp.where(mask, dp, 0.0), axis=1, keepdims=True)
    tp_tok = e_tok / s
    u_col = u_ref[b]
    accept = (tp_tok / dp_tok) >= u_col
    iota8 = jax.lax.broadcasted_iota(jnp.int32, (L, 1), 0)
    n = jnp.min(jnp.where(accept, L, iota8))
    r = jnp.minimum(n, L - 1)

    m_r = jnp.sum(jnp.where(iota8 == r, m, 0.0))
    s_r = jnp.sum(jnp.where(iota8 == r, s, 0.0))
    tl_r = tl_ref[pl.ds(r, 1), :]
    e_r = jnp.exp(tl_r / temp - m_r)
    dp_r = dp_ref[0, pl.ds(r, 1), :]
    padj = jnp.maximum(e_r / s_r - dp_r, TINY)
    S = jnp.sum(padj)
    q_row = q_ref[pl.ds(b, 1), :]
    ratio = (padj / S) / q_row
    mx = jnp.max(ratio)
    iota_v1 = jax.lax.broadcasted_iota(jnp.int32, (1, V), 1)
    rec_at = jnp.min(jnp.where(ratio == mx, iota_v1, V))

    fill = jnp.where(n < L, rec_at, bonus_s[b]).astype(jnp.int32)
    for j in range(L + 1):
        tok_j = tok_s[b, j] if j < L else jnp.int32(0)
        v = jnp.where(j < n, tok_j,
                      jnp.where(j == n, fill, jnp.int32(PLACEHOLDER)))
        out_ref[b, j] = v.astype(jnp.int32)


def kernel(draft_token_ids, cu_num_draft_tokens, draft_probs, target_logits,
           bonus_token_ids, temperature, uniform_probs, q_exp):
    B, L = draft_token_ids.shape
    V = target_logits.shape[-1]
    tok3 = draft_token_ids.reshape(B, L, 1)
    u3 = uniform_probs.reshape(B, L, 1)

    out = pl.pallas_call(
        _body,
        grid=(B,),
        in_specs=[
            pl.BlockSpec(memory_space=pltpu.SMEM),
            pl.BlockSpec(memory_space=pltpu.SMEM),
            pl.BlockSpec(memory_space=pltpu.SMEM),
            pl.BlockSpec((L, V), lambda b: (b, 0)),
            pl.BlockSpec((1, L, V), lambda b: (b, 0, 0)),
            pl.BlockSpec((B, V), lambda b: (0, 0)),
            pl.BlockSpec((B, L, 1), lambda b: (0, 0, 0)),
            pl.BlockSpec((B, L, 1), lambda b: (0, 0, 0)),
        ],
        out_specs=pl.BlockSpec(memory_space=pltpu.SMEM),
        out_shape=jax.ShapeDtypeStruct((B, L + 1), jnp.int32),
        compiler_params=pltpu.CompilerParams(
            dimension_semantics=("parallel",)),
    )(temperature, draft_token_ids, bonus_token_ids,
      target_logits, draft_probs, q_exp, tok3, u3)
    return out


# speedup vs baseline: 1.1861x; 1.1861x over previous
import jax
import jax.numpy as jnp
from jax.experimental import pallas as pl
from jax.experimental.pallas import tpu as pltpu

PLACEHOLDER = -1
TINY = float(jnp.finfo(jnp.float32).tiny)


def _body(temp_s, tok_s, bonus_s, tl_ref, dp_ref, q_ref, u_ref, out_ref):
    L, V = tl_ref.shape
    b = pl.program_id(0)
    temp = temp_s[b]
    tl = tl_ref[...]
    scaled = tl / temp
    m = jnp.max(scaled, axis=1, keepdims=True)
    e = jnp.exp(scaled - m)
    s = jnp.sum(e, axis=1, keepdims=True)

    lane = jax.lax.broadcasted_iota(jnp.int32, (1, 128), 1)
    tl_tok_rows = []
    dp_tok_rows = []
    for l in range(L):
        tok_l = tok_s[b, l]
        base_l = (tok_l // 128) * 128
        sel = lane == (tok_l - base_l)
        tl_win = tl_ref[pl.ds(l, 1), pl.ds(base_l, 128)]
        dp_win = dp_ref[0, pl.ds(l, 1), pl.ds(base_l, 128)]
        tl_tok_rows.append(jnp.sum(jnp.where(sel, tl_win, 0.0)))
        dp_tok_rows.append(jnp.sum(jnp.where(sel, dp_win, 0.0)))
    iota8 = jax.lax.broadcasted_iota(jnp.int32, (L, 1), 0)
    tltok_col = jnp.zeros((L, 1), jnp.float32)
    dptok_col = jnp.zeros((L, 1), jnp.float32)
    for l in range(L):
        tltok_col = jnp.where(iota8 == l, tl_tok_rows[l], tltok_col)
        dptok_col = jnp.where(iota8 == l, dp_tok_rows[l], dptok_col)
    tp_tok = jnp.exp(tltok_col / temp - m) / s
    u_col = u_ref[b]
    accept = (tp_tok / dptok_col) >= u_col
    n = jnp.min(jnp.where(accept, L, iota8))
    r = jnp.minimum(n, L - 1)

    m_r = jnp.sum(jnp.where(iota8 == r, m, 0.0))
    s_r = jnp.sum(jnp.where(iota8 == r, s, 0.0))
    tl_r = tl_ref[pl.ds(r, 1), :]
    e_r = jnp.exp(tl_r / temp - m_r)
    dp_r = dp_ref[0, pl.ds(r, 1), :]
    padj = jnp.maximum(e_r / s_r - dp_r, TINY)
    q_row = q_ref[pl.ds(b, 1), :]
    ratio = padj / q_row
    mx = jnp.max(ratio)
    iota_v1 = jax.lax.broadcasted_iota(jnp.int32, (1, V), 1)
    rec_at = jnp.min(jnp.where(ratio == mx, iota_v1, V))

    fill = jnp.where(n < L, rec_at, bonus_s[b]).astype(jnp.int32)
    for j in range(L + 1):
        tok_j = tok_s[b, j] if j < L else jnp.int32(0)
        v = jnp.where(j < n, tok_j,
                      jnp.where(j == n, fill, jnp.int32(PLACEHOLDER)))
        out_ref[b, j] = v.astype(jnp.int32)


def kernel(draft_token_ids, cu_num_draft_tokens, draft_probs, target_logits,
           bonus_token_ids, temperature, uniform_probs, q_exp):
    B, L = draft_token_ids.shape
    V = target_logits.shape[-1]
    u3 = uniform_probs.reshape(B, L, 1)

    out = pl.pallas_call(
        _body,
        grid=(B,),
        in_specs=[
            pl.BlockSpec(memory_space=pltpu.SMEM),
            pl.BlockSpec(memory_space=pltpu.SMEM),
            pl.BlockSpec(memory_space=pltpu.SMEM),
            pl.BlockSpec((L, V), lambda b: (b, 0)),
            pl.BlockSpec((1, L, V), lambda b: (b, 0, 0)),
            pl.BlockSpec((B, V), lambda b: (0, 0)),
            pl.BlockSpec((B, L, 1), lambda b: (0, 0, 0)),
        ],
        out_specs=pl.BlockSpec(memory_space=pltpu.SMEM),
        out_shape=jax.ShapeDtypeStruct((B, L + 1), jnp.int32),
        compiler_params=pltpu.CompilerParams(
            dimension_semantics=("parallel",)),
    )(temperature, draft_token_ids, bonus_token_ids,
      target_logits, draft_probs, q_exp, u3)
    return out


# speedup vs baseline: 1.3449x; 1.1338x over previous
import jax
import jax.numpy as jnp
from jax.experimental import pallas as pl
from jax.experimental.pallas import tpu as pltpu

PLACEHOLDER = -1
TINY = float(jnp.finfo(jnp.float32).tiny)
G = 2


def _body(temp_s, tok_s, bonus_s, tl_ref, dp_ref, q_ref, u_ref, out_ref):
    R, V = tl_ref.shape
    L = R // G
    step = pl.program_id(0)

    iotaR = jax.lax.broadcasted_iota(jnp.int32, (R, 1), 0)
    temp_col = jnp.zeros((R, 1), jnp.float32)
    temps = []
    for g in range(G):
        t_g = temp_s[step * G + g]
        temps.append(t_g)
        temp_col = jnp.where(iotaR // L == g, t_g, temp_col)

    tl = tl_ref[...]
    scaled = tl / temp_col
    m = jnp.max(scaled, axis=1, keepdims=True)
    e = jnp.exp(scaled - m)
    s = jnp.sum(e, axis=1, keepdims=True)

    lane = jax.lax.broadcasted_iota(jnp.int32, (1, 128), 1)
    iota8 = jax.lax.broadcasted_iota(jnp.int32, (L, 1), 0)
    for g in range(G):
        b = step * G + g
        tl_tok_vals = []
        dp_tok_vals = []
        for l in range(L):
            tok_l = tok_s[b, l]
            base_l = (tok_l // 128) * 128
            sel = lane == (tok_l - base_l)
            tl_win = tl_ref[pl.ds(g * L + l, 1), pl.ds(base_l, 128)]
            dp_win = dp_ref[g, pl.ds(l, 1), pl.ds(base_l, 128)]
            tl_tok_vals.append(jnp.sum(jnp.where(sel, tl_win, 0.0)))
            dp_tok_vals.append(jnp.sum(jnp.where(sel, dp_win, 0.0)))
        tltok_col = jnp.zeros((L, 1), jnp.float32)
        dptok_col = jnp.zeros((L, 1), jnp.float32)
        for l in range(L):
            tltok_col = jnp.where(iota8 == l, tl_tok_vals[l], tltok_col)
            dptok_col = jnp.where(iota8 == l, dp_tok_vals[l], dptok_col)

        m_g = m[g * L:(g + 1) * L]
        s_g = s[g * L:(g + 1) * L]
        tp_tok = jnp.exp(tltok_col / temps[g] - m_g) / s_g
        u_col = u_ref[b]
        accept = (tp_tok / dptok_col) >= u_col
        n = jnp.min(jnp.where(accept, L, iota8))
        r = jnp.minimum(n, L - 1)

        m_r = jnp.sum(jnp.where(iota8 == r, m_g, 0.0))
        s_r = jnp.sum(jnp.where(iota8 == r, s_g, 0.0))
        tl_r = tl_ref[pl.ds(g * L + r, 1), :]
        e_r = jnp.exp(tl_r / temps[g] - m_r)
        dp_r = dp_ref[g, pl.ds(r, 1), :]
        padj = jnp.maximum(e_r / s_r - dp_r, TINY)
        q_row = q_ref[pl.ds(b, 1), :]
        ratio = padj / q_row
        mx = jnp.max(ratio)
        iota_v1 = jax.lax.broadcasted_iota(jnp.int32, (1, V), 1)
        rec_at = jnp.min(jnp.where(ratio == mx, iota_v1, V))

        fill = jnp.where(n < L, rec_at, bonus_s[b]).astype(jnp.int32)
        for j in range(L + 1):
            tok_j = tok_s[b, j] if j < L else jnp.int32(0)
            v = jnp.where(j < n, tok_j,
                          jnp.where(j == n, fill, jnp.int32(PLACEHOLDER)))
            out_ref[b, j] = v.astype(jnp.int32)


def kernel(draft_token_ids, cu_num_draft_tokens, draft_probs, target_logits,
           bonus_token_ids, temperature, uniform_probs, q_exp):
    B, L = draft_token_ids.shape
    V = target_logits.shape[-1]
    u3 = uniform_probs.reshape(B, L, 1)

    out = pl.pallas_call(
        _body,
        grid=(B // G,),
        in_specs=[
            pl.BlockSpec(memory_space=pltpu.SMEM),
            pl.BlockSpec(memory_space=pltpu.SMEM),
            pl.BlockSpec(memory_space=pltpu.SMEM),
            pl.BlockSpec((G * L, V), lambda i: (i, 0)),
            pl.BlockSpec((G, L, V), lambda i: (i, 0, 0)),
            pl.BlockSpec((B, V), lambda i: (0, 0)),
            pl.BlockSpec((B, L, 1), lambda i: (0, 0, 0)),
        ],
        out_specs=pl.BlockSpec(memory_space=pltpu.SMEM),
        out_shape=jax.ShapeDtypeStruct((B, L + 1), jnp.int32),
        compiler_params=pltpu.CompilerParams(
            dimension_semantics=("parallel",)),
    )(temperature, draft_token_ids, bonus_token_ids,
      target_logits, draft_probs, q_exp, u3)
    return out
